# baseline (speedup 1.0000x reference)
import functools

import jax
import jax.numpy as jnp
from jax import lax
from jax.experimental import pallas as pl
from jax.experimental.pallas import tpu as pltpu
from jax.experimental.pallas import tpu_sc as plsc

_D = 64
_NW = 32
_S = 200
_B = 4096
_UNIT = 256
_UNITS_PER_W = (_S * _B // _UNIT) // _NW
_PLANE = _B * _D


def _make_gather():
    mesh = plsc.VectorSubcoreMesh(core_axis_name="c", subcore_axis_name="s")

    @functools.partial(
        pl.kernel,
        mesh=mesh,
        out_type=jax.ShapeDtypeStruct((_S, _PLANE), jnp.float32),
        scratch_types=[
            [pltpu.VMEM((_UNIT,), jnp.int32)] * 2,
            [pltpu.VMEM((_UNIT, _D), jnp.float32)] * 2,
            [pltpu.VMEM((8, 2048), jnp.float32)] * 2,
            [pltpu.SemaphoreType.DMA] * 2,
            [pltpu.SemaphoreType.DMA] * 2,
        ],
        compiler_params=pltpu.CompilerParams(
            use_tc_tiling_on_sc=False, needs_layout_passes=False
        ),
    )
    def gather(ids_hbm, table_hbm, out_hbm, idx_v, rows_v, stage_v, gsems, wsems):
        wid = lax.axis_index("s") * 2 + lax.axis_index("c")
        u0 = wid * _UNITS_PER_W
        iota = lax.iota(jnp.int32, 16)
        rows16 = [iota + k * 16 for k in range(16)]

        def unit_pos(u):
            g = u0 + u
            return g // 16, g % 16

        def stage_and_fire(u, b):
            s, q = unit_pos(u)
            pltpu.sync_copy(
                ids_hbm.at[pl.ds(s * _B + q * _UNIT, _UNIT)], idx_v[b]
            )
            pltpu.async_copy(table_hbm.at[idx_v[b]], rows_v[b], gsems[b])

        stage_and_fire(0, 0)
        stage_and_fire(1, 1)

        def transpose_unit(b):
            @plsc.parallel_loop(0, 8)
            def _(dg):
                colbase = jnp.full((16,), dg * 8, jnp.int32)
                for ds in range(8):
                    col = colbase + ds
                    vs = [
                        plsc.load_gather(rows_v[b], [rows16[k], col])
                        for k in range(16)
                    ]
                    for k in range(16):
                        stage_v[b][
                            dg,
                            pl.ds((k // 8) * 1024 + ds * 128 + (k % 8) * 16, 16),
                        ] = vs[k]

        def writeback(u, b):
            s, q = unit_pos(u)
            for dg in range(8):
                pltpu.async_copy(
                    stage_v[b].at[dg],
                    out_hbm.at[s, pl.ds(dg * 32768 + q * 2048, 2048)],
                    wsems[b],
                )

        def wb_drain(b):
            for dg in range(8):
                pltpu.make_async_copy(
                    stage_v[b].at[dg],
                    out_hbm.at[0, pl.ds(dg * 32768, 2048)],
                    wsems[b],
                ).wait()

        def body(grp, carry):
            for b in range(2):
                u = grp * 2 + b
                pltpu.make_async_copy(
                    table_hbm.at[idx_v[b]], rows_v[b], gsems[b]
                ).wait()

                @pl.when(grp >= 1)
                def _():
                    wb_drain(b)

                transpose_unit(b)
                writeback(u, b)

                @pl.when(u + 2 < _UNITS_PER_W)
                def _():
                    stage_and_fire(u + 2, b)

            return carry

        lax.fori_loop(0, _UNITS_PER_W // 2, body, 0)
        wb_drain(0)
        wb_drain(1)

    return gather


def kernel(token_ids, table):
    ids = token_ids.T.reshape(-1)
    out = _make_gather()(ids, table)
    return (
        out.reshape(_S, 8, 32, 8, 128)
        .transpose(2, 4, 0, 1, 3)
        .reshape(_B, _S, _D)
    )


# speedup vs baseline: 1.0030x; 1.0030x over previous
import functools

import jax
import jax.numpy as jnp
from jax import lax
from jax.experimental import pallas as pl
from jax.experimental.pallas import tpu as pltpu
from jax.experimental.pallas import tpu_sc as plsc

_D = 64
_NW = 32
_S = 200
_B = 4096
_UNIT = 256
_UNITS_PER_W = (_S * _B // _UNIT) // _NW
_PLANE = _B * _D


def _make_gather():
    mesh = plsc.VectorSubcoreMesh(core_axis_name="c", subcore_axis_name="s")

    @functools.partial(
        pl.kernel,
        mesh=mesh,
        out_type=jax.ShapeDtypeStruct((_S, _PLANE), jnp.float32),
        scratch_types=[
            [pltpu.VMEM((_UNIT,), jnp.int32)] * 2,
            [pltpu.VMEM((_UNIT, _D), jnp.float32)] * 2,
            [pltpu.VMEM((8, 2048), jnp.float32)] * 2,
            [pltpu.SemaphoreType.DMA] * 2,
            [pltpu.SemaphoreType.DMA] * 2,
        ],
        compiler_params=pltpu.CompilerParams(
            use_tc_tiling_on_sc=False, needs_layout_passes=False,
            disable_bounds_checks=True
        ),
    )
    def gather(ids_hbm, table_hbm, out_hbm, idx_v, rows_v, stage_v, gsems, wsems):
        wid = lax.axis_index("s") * 2 + lax.axis_index("c")
        u0 = wid * _UNITS_PER_W
        iota = lax.iota(jnp.int32, 16)
        rows16 = [iota + k * 16 for k in range(16)]

        def unit_pos(u):
            g = u0 + u
            return g // 16, g % 16

        def stage_and_fire(u, b):
            s, q = unit_pos(u)
            pltpu.sync_copy(
                ids_hbm.at[pl.ds(s * _B + q * _UNIT, _UNIT)], idx_v[b]
            )
            pltpu.async_copy(table_hbm.at[idx_v[b]], rows_v[b], gsems[b])

        stage_and_fire(0, 0)
        stage_and_fire(1, 1)

        def transpose_unit(b):
            @plsc.parallel_loop(0, 8)
            def _(dg):
                colbase = jnp.full((16,), dg * 8, jnp.int32)
                for ds in range(8):
                    col = colbase + ds
                    vs = [
                        plsc.load_gather(rows_v[b], [rows16[k], col])
                        for k in range(16)
                    ]
                    for k in range(16):
                        stage_v[b][
                            dg,
                            pl.ds((k // 8) * 1024 + ds * 128 + (k % 8) * 16, 16),
                        ] = vs[k]

        def writeback(u, b):
            s, q = unit_pos(u)
            for dg in range(8):
                pltpu.async_copy(
                    stage_v[b].at[dg],
                    out_hbm.at[s, pl.ds(dg * 32768 + q * 2048, 2048)],
                    wsems[b],
                )

        def wb_drain(b):
            for dg in range(8):
                pltpu.make_async_copy(
                    stage_v[b].at[dg],
                    out_hbm.at[0, pl.ds(dg * 32768, 2048)],
                    wsems[b],
                ).wait()

        def body(grp, carry):
            for b in range(2):
                u = grp * 2 + b
                pltpu.make_async_copy(
                    table_hbm.at[idx_v[b]], rows_v[b], gsems[b]
                ).wait()

                @pl.when(grp >= 1)
                def _():
                    wb_drain(b)

                transpose_unit(b)
                writeback(u, b)

                @pl.when(u + 2 < _UNITS_PER_W)
                def _():
                    stage_and_fire(u + 2, b)

            return carry

        lax.fori_loop(0, _UNITS_PER_W // 2, body, 0)
        wb_drain(0)
        wb_drain(1)

    return gather


def kernel(token_ids, table):
    ids = token_ids.T.reshape(-1)
    out = _make_gather()(ids, table)
    return (
        out.reshape(_S, 8, 32, 8, 128)
        .transpose(2, 4, 0, 1, 3)
        .reshape(_B, _S, _D)
    )


# speedup vs baseline: 1.8021x; 1.7967x over previous
import functools

import jax
import jax.numpy as jnp
from jax import lax
from jax.experimental import pallas as pl
from jax.experimental.pallas import tpu as pltpu
from jax.experimental.pallas import tpu_sc as plsc

_D = 64
_NW = 32
_S = 200
_B = 4096
_UNIT = 256
_UNITS_PER_W = (_S * _B // _UNIT) // _NW
_PLANE = _B * _D


def _make_gather():
    mesh = plsc.VectorSubcoreMesh(core_axis_name="c", subcore_axis_name="s")

    @functools.partial(
        pl.kernel,
        mesh=mesh,
        out_type=jax.ShapeDtypeStruct((_S, _PLANE), jnp.float32),
        scratch_types=[
            [pltpu.VMEM((_UNIT,), jnp.int32)] * 2,
            [pltpu.VMEM((_UNIT, _D), jnp.float32)] * 2,
            [pltpu.VMEM((16384,), jnp.float32)] * 2,
            [pltpu.SemaphoreType.DMA] * 2,
            [pltpu.SemaphoreType.DMA] * 2,
        ],
        compiler_params=pltpu.CompilerParams(
            use_tc_tiling_on_sc=False, needs_layout_passes=False,
            disable_bounds_checks=True
        ),
    )
    def gather(ids_hbm, table_hbm, out_hbm, idx_v, rows_v, stage_v, gsems, wsems):
        wid = lax.axis_index("s") * 2 + lax.axis_index("c")
        u0 = wid * _UNITS_PER_W
        iota = lax.iota(jnp.int32, 16)
        rows16 = [iota + k * 16 for k in range(16)]
        rot = [(c0 + iota) % 16 for c0 in range(16)]
        spat = [(r // 8) * 2048 + (r % 8) * 128 + iota for r in rot]

        def unit_pos(u):
            g = u0 + u
            return g // 16, g % 16

        def stage_and_fire(u, b):
            s, q = unit_pos(u)
            pltpu.sync_copy(
                ids_hbm.at[pl.ds(s * _B + q * _UNIT, _UNIT)], idx_v[b]
            )
            pltpu.async_copy(table_hbm.at[idx_v[b]], rows_v[b], gsems[b])

        stage_and_fire(0, 0)
        stage_and_fire(1, 1)

        def transpose_unit(b):
            @plsc.parallel_loop(0, 16)
            def _(k):
                rowk = iota + k * 16
                kbase = (k // 8) * 1024 + (k % 8) * 16
                for d0 in (0, 16, 32, 48):
                    for c0 in range(16):
                        v = plsc.load_gather(
                            rows_v[b], [rowk, rot[c0] + d0]
                        )
                        plsc.store_scatter(
                            stage_v[b],
                            [spat[c0] + ((d0 // 8) * 2048 + kbase)],
                            v,
                        )

        def writeback(u, b):
            s, q = unit_pos(u)
            for dg in range(8):
                pltpu.async_copy(
                    stage_v[b].at[pl.ds(dg * 2048, 2048)],
                    out_hbm.at[s, pl.ds(dg * 32768 + q * 2048, 2048)],
                    wsems[b],
                )

        def wb_drain(b):
            for dg in range(8):
                pltpu.make_async_copy(
                    stage_v[b].at[pl.ds(dg * 2048, 2048)],
                    out_hbm.at[0, pl.ds(dg * 32768, 2048)],
                    wsems[b],
                ).wait()

        def body(grp, carry):
            for b in range(2):
                u = grp * 2 + b
                pltpu.make_async_copy(
                    table_hbm.at[idx_v[b]], rows_v[b], gsems[b]
                ).wait()

                @pl.when(grp >= 1)
                def _():
                    wb_drain(b)

                transpose_unit(b)
                writeback(u, b)

                @pl.when(u + 2 < _UNITS_PER_W)
                def _():
                    stage_and_fire(u + 2, b)

            return carry

        lax.fori_loop(0, _UNITS_PER_W // 2, body, 0)
        wb_drain(0)
        wb_drain(1)

    return gather


def kernel(token_ids, table):
    ids = token_ids.T.reshape(-1)
    out = _make_gather()(ids, table)
    return (
        out.reshape(_S, 8, 32, 8, 128)
        .transpose(2, 4, 0, 1, 3)
        .reshape(_B, _S, _D)
    )
